# baseline (speedup 1.0000x reference)
import functools

import jax
import jax.numpy as jnp
from jax import lax
from jax.experimental import pallas as pl
from jax.experimental.pallas import tpu as pltpu
from jax.experimental.pallas import tpu_sc as plsc

NCLS = 19
NPIX = 8 * 512 * 512
PLANE = 512 * 512
LANES = 16
NWORK = 32
NPW = NPIX // NWORK
CHUNK = 16384
NCHUNK = NPW // CHUNK
OBINS = 4096
BSHIFT = 23 - 13
BBASE = 128 << 13


def _hist_body(x_hbm, t_hbm, out_hbm, lbl_v, pbuf_v, hist_v, outbuf_v,
               sem0, sem1, sem2):
    wid = lax.axis_index("s") * 2 + lax.axis_index("c")
    pix_base = wid * NPW
    batch = wid // 4
    inb = (wid % 4) * NPW

    pltpu.sync_copy(t_hbm.at[wid], lbl_v)

    izeros = jnp.zeros((LANES,), jnp.int32)
    ione = jnp.ones((LANES,), jnp.int32)

    def _zero_body(j, cval):
        hist_v[pl.ds(j * LANES, LANES)] = izeros
        return cval

    zdep = plsc.parallel_loop(0, (2 * OBINS) // LANES, unroll=8,
                              carry=jnp.int32(0))(_zero_body)

    TOTAL = NCLS * NCHUNK

    def addr(s):
        cc = s >> 2
        kk = s & (NCHUNK - 1)
        return pl.multiple_of(
            (batch * NCLS + cc) * PLANE + inb + kk * CHUNK, CHUNK)

    buf0 = pbuf_v.at[pl.ds(0, CHUNK)]
    buf1 = pbuf_v.at[pl.ds(CHUNK, CHUNK)]

    def start(s, buf, sem):
        pltpu.async_copy(x_hbm.at[pl.ds(addr(s), CHUNK)], buf, sem)

    def wait(s, buf, sem):
        pltpu.make_async_copy(x_hbm.at[pl.ds(addr(s), CHUNK)], buf,
                              sem).wait()

    def compute_chunk(s, bufbase):
        loff = (s & (NCHUNK - 1)) * CHUNK
        cval_cls = s >> 2

        def _main(i, cval):
            o = i * LANES
            p = pbuf_v[pl.ds(bufbase + o, LANES)]
            lbl = lbl_v[pl.ds(loff + o, LANES)]
            m = lbl == cval_cls
            q = jnp.where(m, 4.0 - p, 2.0 + p)
            bits = plsc.bitcast(q, jnp.int32)
            bin_ = (bits >> BSHIFT) - BBASE
            plsc.addupdate_scatter(hist_v, [bin_], ione)
            return cval

        return plsc.parallel_loop(0, CHUNK // LANES, unroll=8,
                                  carry=jnp.int32(0))(_main)

    def _reduce_body(j, cval):
        o = j * LANES
        bg = hist_v[pl.ds(o, LANES)]
        hist_v[pl.ds(o, LANES)] = izeros
        fg = hist_v[pl.ds(OBINS + o, LANES)]
        hist_v[pl.ds(OBINS + o, LANES)] = izeros
        outbuf_v[pl.ds(o, LANES)] = bg + fg
        outbuf_v[pl.ds(OBINS + o, LANES)] = fg
        return cval

    def boundary(s, dep):
        cc = s >> 2

        @pl.when((s & (NCHUNK - 1)) == NCHUNK - 1)
        def _():
            @pl.when(cc > 0)
            def _():
                pltpu.make_async_copy(outbuf_v, out_hbm.at[0], sem2).wait()

            rdep = plsc.parallel_loop(0, OBINS // LANES, unroll=2,
                                      carry=dep)(_reduce_body)
            base = wid * NCLS + cc + jnp.minimum(rdep, 0)
            pltpu.async_copy(outbuf_v, out_hbm.at[base], sem2)

    start(0, buf0, sem0)
    start(1, buf1, sem1)

    def step2_body(s2, carry):
        s0 = s2 * 2
        s1 = s0 + 1
        wait(s0, buf0, sem0)
        carry = carry + compute_chunk(s0, 0)

        @pl.when(s0 + 2 < TOTAL)
        def _():
            start(s0 + 2, buf0, sem0)

        wait(s1, buf1, sem1)
        carry = carry + compute_chunk(s1, CHUNK)

        @pl.when(s1 + 2 < TOTAL)
        def _():
            start(s1 + 2, buf1, sem1)

        boundary(s1, carry)
        return carry

    lax.fori_loop(0, TOTAL // 2, step2_body, zdep)

    pltpu.make_async_copy(outbuf_v, out_hbm.at[0], sem2).wait()


_hist_call = functools.partial(
    pl.kernel,
    out_type=jax.ShapeDtypeStruct((NWORK * NCLS, 2 * OBINS), jnp.int32),
    mesh=plsc.VectorSubcoreMesh(core_axis_name="c", subcore_axis_name="s"),
    compiler_params=pltpu.CompilerParams(needs_layout_passes=False),
    scratch_types=[
        pltpu.VMEM((NPW,), jnp.int32),
        pltpu.VMEM((2 * CHUNK,), jnp.float32),
        pltpu.VMEM((2 * OBINS + 16,), jnp.int32),
        pltpu.VMEM((2 * OBINS,), jnp.int32),
        pltpu.SemaphoreType.DMA,
        pltpu.SemaphoreType.DMA,
        pltpu.SemaphoreType.DMA,
    ],
)(_hist_body)


def _scan_body(cnt_ref, pos_ref, out_ref):
    cnt = jnp.sum(cnt_ref[...], axis=0).astype(jnp.float32)
    pos = jnp.sum(pos_ref[...], axis=0).astype(jnp.float32)

    def rcum(x):
        y = x
        s = 1
        while s < OBINS:
            shifted = jnp.concatenate(
                [y[:, s:], jnp.zeros((NCLS, s), jnp.float32)], axis=1)
            y = y + shifted
            s *= 2
        return y

    n_incl = rcum(cnt)
    p_incl = rcum(pos)
    n_excl = n_incl - cnt
    p_excl = p_incl - pos
    g = p_incl[:, 0:1]

    def jac(n, p):
        return 1.0 - (g - p) / jnp.maximum(g + n - p, 1.0)

    emid = (lax.broadcasted_iota(jnp.int32, (NCLS, OBINS), 1).astype(
        jnp.float32) + 0.5) * (1.0 / OBINS)
    losses = jnp.sum(emid * (jac(n_incl, p_incl) - jac(n_excl, p_excl)),
                     axis=1, keepdims=True)
    present = (g > 0.0).astype(jnp.float32)
    total = jnp.sum(losses * present) / jnp.maximum(jnp.sum(present), 1.0)
    out_ref[...] = jnp.reshape(total, (1, 1))


_scan_call = pl.pallas_call(
    _scan_body,
    out_shape=jax.ShapeDtypeStruct((1, 1), jnp.float32),
)


def kernel(inputs, target):
    x = inputs.reshape(-1)
    t = target.reshape(NWORK, NPW)
    parts = _hist_call(x, t).reshape(NWORK, NCLS, 2, OBINS)
    out = _scan_call(parts[:, :, 0, :], parts[:, :, 1, :])
    return out[0, 0]


# speedup vs baseline: 1.1637x; 1.1637x over previous
import functools

import jax
import jax.numpy as jnp
from jax import lax
from jax.experimental import pallas as pl
from jax.experimental.pallas import tpu as pltpu
from jax.experimental.pallas import tpu_sc as plsc

NCLS = 19
NPIX = 8 * 512 * 512
PLANE = 512 * 512
LANES = 16
NWORK = 32
NPW = NPIX // NWORK
CHUNK = 16384
NCHUNK = NPW // CHUNK
NBINS = 2048
BSHIFT = 23 - 11


def _hist_body(x_hbm, t_hbm, out_hbm, lbl_v, pbuf_v, hist_v, outbuf_v,
               sem0, sem1, sem2):
    wid = lax.axis_index("s") * 2 + lax.axis_index("c")
    batch = wid // 4
    inb = (wid % 4) * NPW

    pltpu.sync_copy(t_hbm.at[wid], lbl_v)

    izeros = jnp.zeros((LANES,), jnp.int32)
    ipos = jnp.full((LANES,), 8193, jnp.int32)
    ione = jnp.ones((LANES,), jnp.int32)

    def _zero_body(j, cval):
        hist_v[pl.ds(j * LANES, LANES)] = izeros
        return cval

    zdep = plsc.parallel_loop(0, NBINS // LANES, unroll=8,
                              carry=jnp.int32(0))(_zero_body)

    TOTAL = NCLS * NCHUNK

    def addr(s):
        cc = s >> 2
        kk = s & (NCHUNK - 1)
        return pl.multiple_of(
            (batch * NCLS + cc) * PLANE + inb + kk * CHUNK, CHUNK)

    buf0 = pbuf_v.at[pl.ds(0, CHUNK)]
    buf1 = pbuf_v.at[pl.ds(CHUNK, CHUNK)]

    def start(s, buf, sem):
        pltpu.async_copy(x_hbm.at[pl.ds(addr(s), CHUNK)], buf, sem)

    def wait(s, buf, sem):
        pltpu.make_async_copy(x_hbm.at[pl.ds(addr(s), CHUNK)], buf,
                              sem).wait()

    def compute_chunk(s, bufbase):
        loff = (s & (NCHUNK - 1)) * CHUNK
        cval_cls = s >> 2

        def _main(i, cval):
            o = i * LANES
            p = pbuf_v[pl.ds(bufbase + o, LANES)]
            lbl = lbl_v[pl.ds(loff + o, LANES)]
            m = lbl == cval_cls
            q = jnp.where(m, 2.0 - p, 1.0 + p)
            bits = plsc.bitcast(q, jnp.int32)
            bin_ = (bits >> BSHIFT) - (127 << (23 - BSHIFT))
            val = jnp.where(m, ipos, ione)
            plsc.addupdate_scatter(hist_v, [bin_], val)
            return cval

        return plsc.parallel_loop(0, CHUNK // LANES, unroll=8,
                                  carry=jnp.int32(0))(_main)

    def _reduce_body(j, cval):
        o = j * LANES
        v = hist_v[pl.ds(o, LANES)]
        hist_v[pl.ds(o, LANES)] = izeros
        outbuf_v[pl.ds(o, LANES)] = v & 8191
        outbuf_v[pl.ds(NBINS + o, LANES)] = v >> 13
        return cval

    def boundary(s, dep):
        cc = s >> 2

        @pl.when((s & (NCHUNK - 1)) == NCHUNK - 1)
        def _():
            @pl.when(cc > 0)
            def _():
                pltpu.make_async_copy(outbuf_v, out_hbm.at[0], sem2).wait()

            rdep = plsc.parallel_loop(0, NBINS // LANES, unroll=2,
                                      carry=dep)(_reduce_body)
            base = wid * NCLS + cc + jnp.minimum(rdep, 0)
            pltpu.async_copy(outbuf_v, out_hbm.at[base], sem2)

    start(0, buf0, sem0)
    start(1, buf1, sem1)

    def step2_body(s2, carry):
        s0 = s2 * 2
        s1 = s0 + 1
        wait(s0, buf0, sem0)
        carry = carry + compute_chunk(s0, 0)

        @pl.when(s0 + 2 < TOTAL)
        def _():
            start(s0 + 2, buf0, sem0)

        wait(s1, buf1, sem1)
        carry = carry + compute_chunk(s1, CHUNK)

        @pl.when(s1 + 2 < TOTAL)
        def _():
            start(s1 + 2, buf1, sem1)

        boundary(s1, carry)
        return carry

    lax.fori_loop(0, TOTAL // 2, step2_body, zdep)

    pltpu.make_async_copy(outbuf_v, out_hbm.at[0], sem2).wait()


_hist_call = functools.partial(
    pl.kernel,
    out_type=jax.ShapeDtypeStruct((NWORK * NCLS, 2 * NBINS), jnp.int32),
    mesh=plsc.VectorSubcoreMesh(core_axis_name="c", subcore_axis_name="s"),
    compiler_params=pltpu.CompilerParams(needs_layout_passes=False),
    scratch_types=[
        pltpu.VMEM((NPW,), jnp.int32),
        pltpu.VMEM((2 * CHUNK,), jnp.float32),
        pltpu.VMEM((NBINS + 16,), jnp.int32),
        pltpu.VMEM((2 * NBINS,), jnp.int32),
        pltpu.SemaphoreType.DMA,
        pltpu.SemaphoreType.DMA,
        pltpu.SemaphoreType.DMA,
    ],
)(_hist_body)


def _scan_body(cnt_ref, pos_ref, out_ref):
    cnt = jnp.sum(cnt_ref[...], axis=0).astype(jnp.float32)
    pos = jnp.sum(pos_ref[...], axis=0).astype(jnp.float32)

    def rcum(x):
        y = x
        s = 1
        while s < NBINS:
            shifted = jnp.concatenate(
                [y[:, s:], jnp.zeros((NCLS, s), jnp.float32)], axis=1)
            y = y + shifted
            s *= 2
        return y

    n_incl = rcum(cnt)
    p_incl = rcum(pos)
    n_excl = n_incl - cnt
    p_excl = p_incl - pos
    g = p_incl[:, 0:1]

    def jac(n, p):
        return 1.0 - (g - p) / jnp.maximum(g + n - p, 1.0)

    emid = (lax.broadcasted_iota(jnp.int32, (NCLS, NBINS), 1).astype(
        jnp.float32) + 0.5) * (1.0 / NBINS)
    losses = jnp.sum(emid * (jac(n_incl, p_incl) - jac(n_excl, p_excl)),
                     axis=1, keepdims=True)
    present = (g > 0.0).astype(jnp.float32)
    total = jnp.sum(losses * present) / jnp.maximum(jnp.sum(present), 1.0)
    out_ref[...] = jnp.reshape(total, (1, 1))


_scan_call = pl.pallas_call(
    _scan_body,
    out_shape=jax.ShapeDtypeStruct((1, 1), jnp.float32),
)


def kernel(inputs, target):
    x = inputs.reshape(-1)
    t = target.reshape(NWORK, NPW)
    parts = _hist_call(x, t).reshape(NWORK, NCLS, 2, NBINS)
    out = _scan_call(parts[:, :, 0, :], parts[:, :, 1, :])
    return out[0, 0]
